# baseline (device time: 50592 ns/iter reference)
import jax
import jax.numpy as jnp
from jax import lax
from jax.experimental import pallas as pl
from jax.experimental.pallas import tpu as pltpu

NC = 8


def kernel(x, dy):
    k_per, d = x.shape
    _, f = dy.shape
    half = d // 2
    fhalf = f // 2
    fc = fhalf // NC

    def body(
        x_ref, dy_ref, out_ref,
        send_x, recv_x, send_y, recv_y,
        sx_sems, rx_sems, sy_sems, ry_sems,
    ):
        ix = lax.axis_index("x")
        iy = lax.axis_index("y")
        iz = lax.axis_index("z")
        px = 1 - ix
        py = iy ^ 1
        h = iy & 1

        barrier_sem = pltpu.get_barrier_semaphore()
        pl.semaphore_signal(
            barrier_sem, inc=1,
            device_id=(px, iy, iz), device_id_type=pl.DeviceIdType.MESH,
        )
        pl.semaphore_signal(
            barrier_sem, inc=1,
            device_id=(ix, py, iz), device_id_type=pl.DeviceIdType.MESH,
        )

        x_p = x_ref[:, pl.ds(px * half, half)].astype(jnp.bfloat16)
        x_m = x_ref[:, pl.ds(ix * half, half)].astype(jnp.bfloat16)

        dims = (((0,), (0,)), ((), ()))
        my_base = h * fhalf
        other_base = (1 - h) * fhalf

        for j in range(NC):
            mycols = pl.ds(my_base + j * fc, fc)
            dyb_j = dy_ref[:, mycols].astype(jnp.bfloat16)
            pp_j = lax.dot_general(
                x_p, dyb_j, dims, preferred_element_type=jnp.float32
            )
            send_x[j] = pp_j.astype(jnp.bfloat16)

        pl.semaphore_wait(barrier_sem, 2)
        x_rdmas = []
        for j in range(NC):
            rdma = pltpu.make_async_remote_copy(
                src_ref=send_x.at[j],
                dst_ref=recv_x.at[j],
                send_sem=sx_sems.at[j],
                recv_sem=rx_sems.at[j],
                device_id=(px, iy, iz),
                device_id_type=pl.DeviceIdType.MESH,
            )
            rdma.start()
            x_rdmas.append(rdma)

        for j in range(NC):
            mycols = pl.ds(my_base + j * fc, fc)
            dyb_j = dy_ref[:, mycols].astype(jnp.bfloat16)
            pm_j = lax.dot_general(
                x_m, dyb_j, dims, preferred_element_type=jnp.float32
            )
            out_ref[:, mycols] = pm_j

        y_rdmas = []
        for j in range(NC):
            mycols = pl.ds(my_base + j * fc, fc)
            x_rdmas[j].wait()
            r_j = out_ref[:, mycols] + recv_x[j].astype(jnp.float32)
            out_ref[:, mycols] = r_j
            send_y[j] = r_j.astype(jnp.bfloat16)
            rdma = pltpu.make_async_remote_copy(
                src_ref=send_y.at[j],
                dst_ref=recv_y.at[j],
                send_sem=sy_sems.at[j],
                recv_sem=ry_sems.at[j],
                device_id=(ix, py, iz),
                device_id_type=pl.DeviceIdType.MESH,
            )
            rdma.start()
            y_rdmas.append(rdma)

        for j in range(NC):
            othercols = pl.ds(other_base + j * fc, fc)
            y_rdmas[j].wait()
            out_ref[:, othercols] = recv_y[j].astype(jnp.float32)

    buf = pltpu.VMEM((NC, half, fc), jnp.bfloat16)
    return pl.pallas_call(
        body,
        out_shape=jax.ShapeDtypeStruct((half, f), jnp.float32),
        in_specs=[
            pl.BlockSpec(memory_space=pltpu.VMEM),
            pl.BlockSpec(memory_space=pltpu.VMEM),
        ],
        out_specs=pl.BlockSpec(memory_space=pltpu.VMEM),
        scratch_shapes=[
            buf, buf, buf, buf,
            pltpu.SemaphoreType.DMA((NC,)),
            pltpu.SemaphoreType.DMA((NC,)),
            pltpu.SemaphoreType.DMA((NC,)),
            pltpu.SemaphoreType.DMA((NC,)),
        ],
        compiler_params=pltpu.CompilerParams(
            collective_id=0, vmem_limit_bytes=100 * 1024 * 1024
        ),
    )(x, dy)


# device time: 43467 ns/iter; 1.1639x vs baseline; 1.1639x over previous
import jax
import jax.numpy as jnp
from jax import lax
from jax.experimental import pallas as pl
from jax.experimental.pallas import tpu as pltpu

NC = 4
NF = NC // 2


def kernel(x, dy):
    k_per, d = x.shape
    _, f = dy.shape
    half = d // 2
    fq = f // 4
    fc = fq // NC

    def body(
        x_ref, dy_ref, out_ref,
        send_x, recv_x, rbuf, recv_y, recv_z, recv_fy, recv_fz,
        sx, rx, sy, ry, sz, rz, sfy, rfy, sfz, rfz,
    ):
        ix = lax.axis_index("x")
        iy = lax.axis_index("y")
        iz = lax.axis_index("z")
        px = 1 - ix
        py = iy ^ 1
        pz = iz ^ 1
        a = iy & 1
        b = iz & 1
        q_me = 2 * a + b
        q_y = 2 * (1 - a) + b
        q_z = 2 * a + (1 - b)
        q_d = 2 * (1 - a) + (1 - b)

        barrier_sem = pltpu.get_barrier_semaphore()
        for dev in ((px, iy, iz), (ix, py, iz), (ix, iy, pz)):
            pl.semaphore_signal(
                barrier_sem, inc=1,
                device_id=dev, device_id_type=pl.DeviceIdType.MESH,
            )

        x_p = x_ref[:, pl.ds(px * half, half)].astype(jnp.bfloat16)
        x_m = x_ref[:, pl.ds(ix * half, half)].astype(jnp.bfloat16)
        dims = (((0,), (0,)), ((), ()))
        my_base = q_me * fq

        dyb_0 = dy_ref[:, pl.ds(my_base, fc)].astype(jnp.bfloat16)
        pp_0 = lax.dot_general(
            x_p, dyb_0, dims, preferred_element_type=jnp.float32
        )
        send_x[0] = pp_0.astype(jnp.bfloat16)

        pl.semaphore_wait(barrier_sem, 3)

        x_rd = []
        for j in range(NC):
            mycols = pl.ds(my_base + j * fc, fc)
            if j > 0:
                dyb_j = dy_ref[:, mycols].astype(jnp.bfloat16)
                pp_j = lax.dot_general(
                    x_p, dyb_j, dims, preferred_element_type=jnp.float32
                )
                send_x[j] = pp_j.astype(jnp.bfloat16)
            rdma = pltpu.make_async_remote_copy(
                src_ref=send_x.at[j],
                dst_ref=recv_x.at[j],
                send_sem=sx.at[j],
                recv_sem=rx.at[j],
                device_id=(px, iy, iz),
                device_id_type=pl.DeviceIdType.MESH,
            )
            rdma.start()
            x_rd.append(rdma)

        for j in range(NC):
            mycols = pl.ds(my_base + j * fc, fc)
            dyb_j = dy_ref[:, mycols].astype(jnp.bfloat16)
            pm_j = lax.dot_general(
                x_m, dyb_j, dims, preferred_element_type=jnp.float32
            )
            out_ref[:, mycols] = pm_j

        y_rd, z_rd = [], []
        for j in range(NC):
            mycols = pl.ds(my_base + j * fc, fc)
            x_rd[j].wait()
            r_j = out_ref[:, mycols] + recv_x[j].astype(jnp.float32)
            out_ref[:, mycols] = r_j
            rbuf[j] = r_j.astype(jnp.bfloat16)
            rd_y = pltpu.make_async_remote_copy(
                src_ref=rbuf.at[j],
                dst_ref=recv_y.at[j],
                send_sem=sy.at[j],
                recv_sem=ry.at[j],
                device_id=(ix, py, iz),
                device_id_type=pl.DeviceIdType.MESH,
            )
            rd_y.start()
            y_rd.append(rd_y)
            rd_z = pltpu.make_async_remote_copy(
                src_ref=rbuf.at[j],
                dst_ref=recv_z.at[j],
                send_sem=sz.at[j],
                recv_sem=rz.at[j],
                device_id=(ix, iy, pz),
                device_id_type=pl.DeviceIdType.MESH,
            )
            rd_z.start()
            z_rd.append(rd_z)

        fy_rd, fz_rd = [], []
        for j in range(NF):
            z_rd[j].wait()
            rdma = pltpu.make_async_remote_copy(
                src_ref=recv_z.at[j],
                dst_ref=recv_fy.at[j],
                send_sem=sfy.at[j],
                recv_sem=rfy.at[j],
                device_id=(ix, py, iz),
                device_id_type=pl.DeviceIdType.MESH,
            )
            rdma.start()
            fy_rd.append(rdma)
        for j in range(NF, NC):
            y_rd[j].wait()
            rdma = pltpu.make_async_remote_copy(
                src_ref=recv_y.at[j],
                dst_ref=recv_fz.at[j - NF],
                send_sem=sfz.at[j - NF],
                recv_sem=rfz.at[j - NF],
                device_id=(ix, iy, pz),
                device_id_type=pl.DeviceIdType.MESH,
            )
            rdma.start()
            fz_rd.append(rdma)

        for j in range(NC):
            if j < NF:
                y_rd[j].wait()
            out_ref[:, pl.ds(q_y * fq + j * fc, fc)] = recv_y[j].astype(
                jnp.float32
            )
        for j in range(NC):
            if j >= NF:
                z_rd[j].wait()
            out_ref[:, pl.ds(q_z * fq + j * fc, fc)] = recv_z[j].astype(
                jnp.float32
            )
        for j in range(NF):
            fy_rd[j].wait()
            out_ref[:, pl.ds(q_d * fq + j * fc, fc)] = recv_fy[j].astype(
                jnp.float32
            )
        for j in range(NF):
            fz_rd[j].wait()
            out_ref[:, pl.ds(q_d * fq + (NF + j) * fc, fc)] = recv_fz[
                j
            ].astype(jnp.float32)

    buf = pltpu.VMEM((NC, half, fc), jnp.bfloat16)
    fbuf = pltpu.VMEM((NF, half, fc), jnp.bfloat16)
    return pl.pallas_call(
        body,
        out_shape=jax.ShapeDtypeStruct((half, f), jnp.float32),
        in_specs=[
            pl.BlockSpec(memory_space=pltpu.VMEM),
            pl.BlockSpec(memory_space=pltpu.VMEM),
        ],
        out_specs=pl.BlockSpec(memory_space=pltpu.VMEM),
        scratch_shapes=[
            buf, buf, buf, buf, buf,
            fbuf, fbuf,
            pltpu.SemaphoreType.DMA((NC,)),
            pltpu.SemaphoreType.DMA((NC,)),
            pltpu.SemaphoreType.DMA((NC,)),
            pltpu.SemaphoreType.DMA((NC,)),
            pltpu.SemaphoreType.DMA((NC,)),
            pltpu.SemaphoreType.DMA((NC,)),
            pltpu.SemaphoreType.DMA((NF,)),
            pltpu.SemaphoreType.DMA((NF,)),
            pltpu.SemaphoreType.DMA((NF,)),
            pltpu.SemaphoreType.DMA((NF,)),
        ],
        compiler_params=pltpu.CompilerParams(
            collective_id=0, vmem_limit_bytes=100 * 1024 * 1024
        ),
    )(x, dy)
